# baseline (speedup 1.0000x reference)
import functools
import math

import jax
import jax.numpy as jnp
from jax import lax
from jax.experimental import pallas as pl
from jax.experimental.pallas import tpu as pltpu
from jax.experimental.pallas import tpu_sc as plsc

H = 32
W = 32
HW = H * W
T = 16384
CK = 64
CV = 512
TOP_K = 30

NEG = -3.0e38
NBINS = 256
BINW = 0.05
INV_BINW = 1.0 / BINW
CAND = 2048
NTILES = 32
QPT = HW // NTILES
RPT = 2 * CV // NTILES
TB = 2048

L = 16


def _sim_body(qkT_ref, qsT_ref, mk_ref, shr_ref, sim_ref, m_ref):
    j = pl.program_id(0)
    qkT = qkT_ref[...]
    qsT = qsT_ref[...]
    mk = mk_ref[...]
    mk2 = mk * mk
    a_sq = jnp.dot(qsT, mk2, preferred_element_type=jnp.float32)
    two_ab = 2.0 * jnp.dot(qsT * qkT, mk, preferred_element_type=jnp.float32)
    b_sq = jnp.sum(qsT * qkT * qkT, axis=1, keepdims=True)
    sim = (-a_sq + two_ab - b_sq) * shr_ref[...] * (1.0 / math.sqrt(CK))
    sim_ref[...] = sim
    bmax = jnp.max(sim, axis=1, keepdims=True)

    @pl.when(j == 0)
    def _():
        m_ref[...] = jnp.full_like(m_ref, NEG)

    m_ref[...] = jnp.maximum(m_ref[...], jnp.broadcast_to(bmax, (HW, L)))


def _tc_similarity(qkT, qsT, mk, shr):
    return pl.pallas_call(
        _sim_body,
        grid=(T // TB,),
        in_specs=[
            pl.BlockSpec((HW, CK), lambda j: (0, 0)),
            pl.BlockSpec((HW, CK), lambda j: (0, 0)),
            pl.BlockSpec((CK, TB), lambda j: (0, j)),
            pl.BlockSpec((1, TB), lambda j: (0, j)),
        ],
        out_specs=[
            pl.BlockSpec((HW, TB), lambda j: (0, j)),
            pl.BlockSpec((HW, L), lambda j: (0, 0)),
        ],
        out_shape=[
            jax.ShapeDtypeStruct((HW, T), jnp.float32),
            jax.ShapeDtypeStruct((HW, L), jnp.float32),
        ],
    )(qkT, qsT, mk, shr)


def _topk_tile(sim_hbm, m_hbm, pack_hbm, row, mbuf, hist, cval, cidx,
               selv, seli, packb, dsem):
    nc = 2
    wid = lax.axis_index("s") * nc + lax.axis_index("c")
    q0 = wid * QPT
    iota = lax.iota(jnp.int32, L)
    lane0 = iota == 0
    onesi = jnp.ones((L,), jnp.int32)
    allt = iota >= 0
    lanebase = iota * NBINS

    pltpu.sync_copy(m_hbm.at[pl.ds(q0 * L, QPT * L)], mbuf)
    pltpu.make_async_copy(sim_hbm.at[q0], row.at[pl.ds(0, T)], dsem).start()

    def zb(c, _):
        hist[pl.ds(c * L, L)] = jnp.zeros((L,), jnp.int32)
        return 0

    lax.fori_loop(0, L * NBINS // L, zb, 0, unroll=8)

    def per_query(i, _):
        pi = i % 2
        rbase = pi * T

        @pl.when(i + 1 < QPT)
        def _():
            pltpu.make_async_copy(sim_hbm.at[q0 + i + 1],
                                  row.at[pl.ds(((i + 1) % 2) * T, T)],
                                  dsem).start()

        pltpu.make_async_copy(sim_hbm.at[q0 + i],
                              row.at[pl.ds(rbase, T)], dsem).wait()
        mv = mbuf[pl.ds(i * L, L)]

        @plsc.parallel_loop(0, T // L, unroll=16)
        def _hist_loop(j):
            v = row[pl.ds(rbase + j * L, L)]
            d = jnp.minimum((mv - v) * INV_BINW, 1.0e6)
            b = jnp.minimum(d.astype(jnp.int32), NBINS - 1) + lanebase
            plsc.addupdate_scatter(hist, [b], onesi, mask=d < float(NBINS - 1))

        zero = jnp.zeros((L,), jnp.int32)
        carry = 0
        bstar = NBINS - 1
        found = False
        bstar_v = jnp.full((L,), NBINS - 1, jnp.int32)
        done_v = jnp.zeros((L,), jnp.int32)
        for c in range(NBINS // L):
            tot = hist[pl.ds(c * L, L)]
            for l in range(1, L):
                tot = tot + hist[pl.ds(l * NBINS + c * L, L)]
            f = plsc.cumsum(tot) + carry
            carry = jnp.max(f)
            cond = (f >= TOP_K) & (done_v == 0)
            cb = jnp.min(jnp.where(cond, iota + c * L, NBINS - 1))
            hit = jnp.max(jnp.where(cond, 1, 0))
            bstar_v = jnp.where((done_v == 0) & (hit > 0),
                                jnp.full((L,), 1, jnp.int32) * cb, bstar_v)
            done_v = done_v | jnp.broadcast_to(hit, (L,))
            for l in range(L):
                hist[pl.ds(l * NBINS + c * L, L)] = zero
        bstar = jnp.max(bstar_v)

        thr = mv - (bstar.astype(jnp.float32) + 1.0) * BINW

        @plsc.parallel_loop(0, T // L, unroll=8,
                            carry=jnp.zeros((L,), jnp.int32))
        def offv(j, off):
            v = row[pl.ds(rbase + j * L, L)]
            msk = v >= thr
            cnt = plsc.all_reduce_population_count(msk)
            pos = plsc.cumsum(jnp.where(msk, jnp.int32(1), jnp.int32(0))) - 1 + off
            ok = msk & (pos < CAND)
            plsc.store_scatter(cval, [pos], v, mask=ok)
            plsc.store_scatter(cidx, [pos], iota + j * L, mask=ok)
            return off + cnt
        m = jnp.minimum(jnp.max(offv), CAND)

        padpos = m + iota
        plsc.store_scatter(cval, [padpos], jnp.full((L,), NEG, jnp.float32),
                           mask=padpos < CAND)
        nv = (m + L - 1) // L

        selv[pl.ds(0, L)] = jnp.full((L,), NEG, jnp.float32)
        selv[pl.ds(L, L)] = jnp.full((L,), NEG, jnp.float32)
        seli[pl.ds(0, L)] = zero
        seli[pl.ds(L, L)] = zero

        def select(s, _):
            def scan(cchunk, st):
                bv, bi, bp = st
                p = cchunk * L + iota
                v = cval[pl.ds(cchunk * L, L)]
                ix = cidx[pl.ds(cchunk * L, L)]
                better = v > bv
                bv = jnp.where(better, v, bv)
                bi = jnp.where(better, ix, bi)
                bp = jnp.where(better, p, bp)
                return bv, bi, bp

            bv0 = jnp.full((L,), NEG, jnp.float32)
            big = jnp.full((L,), jnp.int32(0x7FFFFFFF))
            bv, bi, bp = lax.fori_loop(0, nv, scan, (bv0, big, big))
            mx = jnp.max(bv)
            tied = bv == mx
            tgt = jnp.min(jnp.where(tied, bi, big))
            psel = jnp.min(jnp.where(tied & (bi == tgt), bp, big))
            plsc.store_scatter(selv, [jnp.broadcast_to(s, (L,))],
                               jnp.broadcast_to(mx, (L,)), mask=lane0)
            plsc.store_scatter(seli, [jnp.broadcast_to(s, (L,))],
                               jnp.broadcast_to(tgt, (L,)), mask=lane0)
            plsc.store_scatter(cval, [jnp.broadcast_to(psel, (L,))],
                               jnp.full((L,), NEG, jnp.float32), mask=lane0)
            return 0

        lax.fori_loop(0, TOP_K, select, 0)

        v1 = selv[pl.ds(0, L)]
        v2 = selv[pl.ds(L, L)]
        e1 = jnp.exp(v1)
        e2 = jnp.exp(v2)
        z = jnp.max(plsc.cumsum(e1) + plsc.cumsum(e2))
        w1 = e1 / z
        w2 = e2 / z
        p1 = (plsc.bitcast(w1, jnp.int32) &
              jnp.full((L,), jnp.int32(-65536))) | seli[pl.ds(0, L)]
        p2 = (plsc.bitcast(w2, jnp.int32) &
              jnp.full((L,), jnp.int32(-65536))) | seli[pl.ds(L, L)]
        packb[pl.ds(i * 2 * L, L)] = p1
        packb[pl.ds(i * 2 * L + L, L)] = p2
        return 0

    lax.fori_loop(0, QPT, per_query, 0)
    pltpu.sync_copy(packb, pack_hbm.at[pl.ds(q0 * 2 * L, QPT * 2 * L)])


def _sc_topk(sim, mrep):
    mesh = plsc.VectorSubcoreMesh(core_axis_name="c", subcore_axis_name="s")
    return pl.kernel(
        _topk_tile,
        mesh=mesh,
        compiler_params=pltpu.CompilerParams(needs_layout_passes=False),
        out_type=jax.ShapeDtypeStruct((HW * 2 * L,), jnp.int32),
        scratch_types=[
            pltpu.VMEM((2 * T,), jnp.float32),
            pltpu.VMEM((QPT * L,), jnp.float32),
            pltpu.VMEM((L * NBINS,), jnp.int32),
            pltpu.VMEM((CAND,), jnp.float32),
            pltpu.VMEM((CAND,), jnp.int32),
            pltpu.VMEM((2 * L,), jnp.float32),
            pltpu.VMEM((2 * L,), jnp.int32),
            pltpu.VMEM((QPT * 2 * L,), jnp.int32),
            pltpu.SemaphoreType.DMA,
        ],
    )(sim, mrep)


def _readout_tile(v_hbm, pack_hbm, out_hbm, packb, vbuf, outst, semA, semB):
    nc = 2
    wid = lax.axis_index("s") * nc + lax.axis_index("c")
    r0 = wid * RPT
    iota = lax.iota(jnp.int32, L)
    zero = jnp.zeros((L,), jnp.float32)
    maskidx = jnp.full((L,), jnp.int32(0x3FFF))
    maskw = jnp.full((L,), jnp.int32(-65536))
    sems = [semA, semB]

    pltpu.sync_copy(pack_hbm, packb)

    def start(g, p):
        for r in range(2):
            pltpu.make_async_copy(v_hbm.at[r0 + g * 2 + r],
                                  vbuf.at[pl.ds((p * 2 + r) * T, T)],
                                  sems[p]).start()

    def wait(g, p):
        for r in range(2):
            pltpu.make_async_copy(v_hbm.at[r0 + g * 2 + r],
                                  vbuf.at[pl.ds((p * 2 + r) * T, T)],
                                  sems[p]).wait()

    start(0, 0)
    for g in range(RPT // 2):
        p = g & 1
        if g + 1 < RPT // 2:
            start(g + 1, 1 - p)
        wait(g, p)
        voff0 = (p * 2) * T
        voff1 = (p * 2 + 1) * T

        def qgroup(qg, _, voff0=voff0, voff1=voff1):
            qv32 = (qg * L + iota) * (2 * L)

            def kstep(k, accs, voff0=voff0, voff1=voff1, qv32=qv32):
                a0, b0, a1, b1 = accs
                pk = plsc.load_gather(packb, [qv32 + k])
                ix = pk & maskidx
                w = plsc.bitcast(pk & maskw, jnp.float32)
                g0 = plsc.load_gather(vbuf, [ix + voff0])
                g1 = plsc.load_gather(vbuf, [ix + voff1])
                pk2 = plsc.load_gather(packb, [qv32 + (k + 1)])
                ix2 = pk2 & maskidx
                w2 = plsc.bitcast(pk2 & maskw, jnp.float32)
                h0 = plsc.load_gather(vbuf, [ix2 + voff0])
                h1 = plsc.load_gather(vbuf, [ix2 + voff1])
                return (a0 + w * g0, b0 + w2 * h0, a1 + w * g1, b1 + w2 * h1)

            a0, b0, a1, b1 = plsc.parallel_loop(
                0, TOP_K, step=2, unroll=5,
                carry=(zero, zero, zero, zero))(kstep)
            outst[pl.ds(qg * L, L)] = a0 + b0
            outst[pl.ds(HW + qg * L, L)] = a1 + b1
            return 0

        lax.fori_loop(0, HW // L, qgroup, 0, unroll=3)
        pltpu.sync_copy(outst, out_hbm.at[pl.ds((r0 + g * 2) * HW, 2 * HW)])


def _sc_readout(vflat, pack):
    mesh = plsc.VectorSubcoreMesh(core_axis_name="c", subcore_axis_name="s")
    return pl.kernel(
        _readout_tile,
        mesh=mesh,
        compiler_params=pltpu.CompilerParams(needs_layout_passes=False),
        out_type=jax.ShapeDtypeStruct((2 * CV * HW,), jnp.float32),
        scratch_types=[
            pltpu.VMEM((HW * 2 * L,), jnp.int32),
            pltpu.VMEM((4 * T,), jnp.float32),
            pltpu.VMEM((2 * HW,), jnp.float32),
            pltpu.SemaphoreType.DMA,
            pltpu.SemaphoreType.DMA,
        ],
    )(vflat, pack)


@jax.jit
def kernel(q_key, q_selection, mem_key, mem_shrinkage, mem_value):
    qkT = jnp.swapaxes(q_key.reshape(CK, HW), 0, 1)
    qsT = jnp.swapaxes(q_selection.reshape(CK, HW), 0, 1)
    mk = mem_key.reshape(CK, T)
    shr = mem_shrinkage.reshape(1, T)
    vflat = mem_value.reshape(2 * CV, T)

    sim, mrep = _tc_similarity(qkT, qsT, mk, shr)
    pack = _sc_topk(sim, mrep.reshape(HW * L))
    out = _sc_readout(vflat, pack)
    return out.reshape(2, CV, H, W)


# speedup vs baseline: 1.0654x; 1.0654x over previous
import math

import jax
import jax.numpy as jnp
from jax import lax
from jax.experimental import pallas as pl
from jax.experimental.pallas import tpu as pltpu
from jax.experimental.pallas import tpu_sc as plsc

H = 32
W = 32
HW = H * W
T = 16384
CK = 64
CV = 512
TOP_K = 30

NEG = -3.0e38
NBINS = 256
BINW = 0.05
INV_BINW = 1.0 / BINW
CAND = 2048
NTILES = 32
QPT = HW // NTILES
RPT = 2 * CV // NTILES
TB = 2048

L = 16


def _sim_body(qkT_ref, qsT_ref, mk_ref, shr_ref, sim_ref, m_ref):
    j = pl.program_id(0)
    qkT = qkT_ref[...]
    qsT = qsT_ref[...]
    mk = mk_ref[...]
    mk2 = mk * mk
    a_sq = jnp.dot(qsT, mk2, preferred_element_type=jnp.float32)
    two_ab = 2.0 * jnp.dot(qsT * qkT, mk, preferred_element_type=jnp.float32)
    b_sq = jnp.sum(qsT * qkT * qkT, axis=1, keepdims=True)
    sim = (-a_sq + two_ab - b_sq) * shr_ref[...] * (1.0 / math.sqrt(CK))
    sim_ref[...] = sim
    bmax = jnp.max(sim, axis=1, keepdims=True)

    @pl.when(j == 0)
    def _():
        m_ref[...] = jnp.full_like(m_ref, NEG)

    m_ref[...] = jnp.maximum(m_ref[...], jnp.broadcast_to(bmax, (HW, L)))


def _tc_similarity(qkT, qsT, mk, shr):
    return pl.pallas_call(
        _sim_body,
        grid=(T // TB,),
        in_specs=[
            pl.BlockSpec((HW, CK), lambda j: (0, 0)),
            pl.BlockSpec((HW, CK), lambda j: (0, 0)),
            pl.BlockSpec((CK, TB), lambda j: (0, j)),
            pl.BlockSpec((1, TB), lambda j: (0, j)),
        ],
        out_specs=[
            pl.BlockSpec((HW, TB), lambda j: (0, j)),
            pl.BlockSpec((HW, L), lambda j: (0, 0)),
        ],
        out_shape=[
            jax.ShapeDtypeStruct((HW, T), jnp.float32),
            jax.ShapeDtypeStruct((HW, L), jnp.float32),
        ],
    )(qkT, qsT, mk, shr)


def _topk_tile(sim_hbm, m_hbm, pack_hbm, row, mbuf, hist, cval, cidx,
               selv, seli, packb, dsem):
    nc = 2
    wid = lax.axis_index("s") * nc + lax.axis_index("c")
    q0 = wid * QPT
    iota = lax.iota(jnp.int32, L)
    lane0 = iota == 0
    onesi = jnp.ones((L,), jnp.int32)
    lanebase = iota * NBINS

    pltpu.sync_copy(m_hbm.at[pl.ds(q0 * L, QPT * L)], mbuf)
    pltpu.make_async_copy(sim_hbm.at[q0], row.at[pl.ds(0, T)], dsem).start()

    def zb(c, _):
        hist[pl.ds(c * L, L)] = jnp.zeros((L,), jnp.int32)
        return 0

    lax.fori_loop(0, L * NBINS // L, zb, 0, unroll=8)

    def per_query(i, _):
        pi = i % 2
        rbase = pi * T

        @pl.when(i + 1 < QPT)
        def _():
            pltpu.make_async_copy(sim_hbm.at[q0 + i + 1],
                                  row.at[pl.ds(((i + 1) % 2) * T, T)],
                                  dsem).start()

        pltpu.make_async_copy(sim_hbm.at[q0 + i],
                              row.at[pl.ds(rbase, T)], dsem).wait()
        mv = mbuf[pl.ds(i * L, L)]

        @plsc.parallel_loop(0, T // L, unroll=8)
        def _hist_loop(j):
            v = row[pl.ds(rbase + j * L, L)]
            d = jnp.minimum((mv - v) * INV_BINW, 1.0e6)
            b = jnp.minimum(d.astype(jnp.int32), NBINS - 1) + lanebase
            plsc.addupdate_scatter(hist, [b], onesi, mask=d < float(NBINS - 1))

        zero = jnp.zeros((L,), jnp.int32)
        carry = 0
        bstar_v = jnp.full((L,), NBINS - 1, jnp.int32)
        done_v = jnp.zeros((L,), jnp.int32)
        for c in range(NBINS // L):
            tot = hist[pl.ds(c * L, L)]
            for l in range(1, L):
                tot = tot + hist[pl.ds(l * NBINS + c * L, L)]
            f = plsc.cumsum(tot) + carry
            carry = jnp.max(f)
            cond = (f >= TOP_K) & (done_v == 0)
            cb = jnp.min(jnp.where(cond, iota + c * L, NBINS - 1))
            hit = jnp.max(jnp.where(cond, 1, 0))
            bstar_v = jnp.where((done_v == 0) & (hit > 0),
                                jnp.full((L,), 1, jnp.int32) * cb, bstar_v)
            done_v = done_v | jnp.broadcast_to(hit, (L,))
            for l in range(L):
                hist[pl.ds(l * NBINS + c * L, L)] = zero
        bstar = jnp.max(bstar_v)

        thr = mv - (bstar.astype(jnp.float32) + 1.0) * BINW

        @plsc.parallel_loop(0, T // L, unroll=8,
                            carry=jnp.zeros((L,), jnp.int32))
        def offv(j, off):
            v = row[pl.ds(rbase + j * L, L)]
            msk = v >= thr
            cnt = plsc.all_reduce_population_count(msk)
            pos = plsc.cumsum(jnp.where(msk, jnp.int32(1), jnp.int32(0))) - 1 + off
            ok = msk & (pos < CAND)
            plsc.store_scatter(cval, [pos], v, mask=ok)
            plsc.store_scatter(cidx, [pos], iota + j * L, mask=ok)
            return off + cnt
        m = jnp.minimum(jnp.max(offv), CAND)

        padpos = m + iota
        plsc.store_scatter(cval, [padpos], jnp.full((L,), NEG, jnp.float32),
                           mask=padpos < CAND)
        nv = (m + L - 1) // L

        selv[pl.ds(0, L)] = jnp.full((L,), NEG, jnp.float32)
        selv[pl.ds(L, L)] = jnp.full((L,), NEG, jnp.float32)
        seli[pl.ds(0, L)] = zero
        seli[pl.ds(L, L)] = zero

        def select(s, _):
            def scan(cchunk, st):
                bv, bi, bp = st
                p = cchunk * L + iota
                v = cval[pl.ds(cchunk * L, L)]
                ix = cidx[pl.ds(cchunk * L, L)]
                better = v > bv
                bv = jnp.where(better, v, bv)
                bi = jnp.where(better, ix, bi)
                bp = jnp.where(better, p, bp)
                return bv, bi, bp

            bv0 = jnp.full((L,), NEG, jnp.float32)
            big = jnp.full((L,), jnp.int32(0x7FFFFFFF))
            bv, bi, bp = lax.fori_loop(0, nv, scan, (bv0, big, big))
            mx = jnp.max(bv)
            tied = bv == mx
            tgt = jnp.min(jnp.where(tied, bi, big))
            psel = jnp.min(jnp.where(tied & (bi == tgt), bp, big))
            plsc.store_scatter(selv, [jnp.broadcast_to(s, (L,))],
                               jnp.broadcast_to(mx, (L,)), mask=lane0)
            plsc.store_scatter(seli, [jnp.broadcast_to(s, (L,))],
                               jnp.broadcast_to(tgt, (L,)), mask=lane0)
            plsc.store_scatter(cval, [jnp.broadcast_to(psel, (L,))],
                               jnp.full((L,), NEG, jnp.float32), mask=lane0)
            return 0

        lax.fori_loop(0, TOP_K, select, 0)

        v1 = selv[pl.ds(0, L)]
        v2 = selv[pl.ds(L, L)]
        e1 = jnp.exp(v1)
        e2 = jnp.exp(v2)
        z = jnp.max(plsc.cumsum(e1) + plsc.cumsum(e2))
        w1 = e1 / z
        w2 = e2 / z
        p1 = (plsc.bitcast(w1, jnp.int32) &
              jnp.full((L,), jnp.int32(-65536))) | seli[pl.ds(0, L)]
        p2 = (plsc.bitcast(w2, jnp.int32) &
              jnp.full((L,), jnp.int32(-65536))) | seli[pl.ds(L, L)]
        packb[pl.ds(i * 2 * L, L)] = p1
        packb[pl.ds(i * 2 * L + L, L)] = p2
        return 0

    lax.fori_loop(0, QPT, per_query, 0)
    pltpu.sync_copy(packb, pack_hbm.at[pl.ds(q0 * 2 * L, QPT * 2 * L)])


def _sc_topk(sim, mrep):
    mesh = plsc.VectorSubcoreMesh(core_axis_name="c", subcore_axis_name="s")
    return pl.kernel(
        _topk_tile,
        mesh=mesh,
        compiler_params=pltpu.CompilerParams(needs_layout_passes=False),
        out_type=jax.ShapeDtypeStruct((HW * 2 * L,), jnp.int32),
        scratch_types=[
            pltpu.VMEM((2 * T,), jnp.float32),
            pltpu.VMEM((QPT * L,), jnp.float32),
            pltpu.VMEM((L * NBINS,), jnp.int32),
            pltpu.VMEM((CAND,), jnp.float32),
            pltpu.VMEM((CAND,), jnp.int32),
            pltpu.VMEM((2 * L,), jnp.float32),
            pltpu.VMEM((2 * L,), jnp.int32),
            pltpu.VMEM((QPT * 2 * L,), jnp.int32),
            pltpu.SemaphoreType.DMA,
        ],
    )(sim, mrep)


def _readout_tile(v_hbm, pack_hbm, out_hbm, packb, vbuf, outst, semA, semB):
    nc = 2
    wid = lax.axis_index("s") * nc + lax.axis_index("c")
    r0 = wid * RPT
    iota = lax.iota(jnp.int32, L)
    zero = jnp.zeros((L,), jnp.float32)
    maskidx = jnp.full((L,), jnp.int32(0x3FFF))
    maskw = jnp.full((L,), jnp.int32(-65536))
    sems = [semA, semB]

    pltpu.sync_copy(pack_hbm, packb)

    def start(g, p):
        for r in range(2):
            pltpu.make_async_copy(v_hbm.at[r0 + g * 2 + r],
                                  vbuf.at[pl.ds((p * 2 + r) * T, T)],
                                  sems[p]).start()

    def wait(g, p):
        for r in range(2):
            pltpu.make_async_copy(v_hbm.at[r0 + g * 2 + r],
                                  vbuf.at[pl.ds((p * 2 + r) * T, T)],
                                  sems[p]).wait()

    start(0, 0)
    for g in range(RPT // 2):
        p = g & 1
        if g + 1 < RPT // 2:
            start(g + 1, 1 - p)
        wait(g, p)
        voff0 = (p * 2) * T
        voff1 = (p * 2 + 1) * T

        def qgroup(qg, _, voff0=voff0, voff1=voff1):
            qv32 = (qg * L + iota) * (2 * L)

            def kstep(k, accs, voff0=voff0, voff1=voff1, qv32=qv32):
                a0, b0, a1, b1 = accs
                pk = plsc.load_gather(packb, [qv32 + k])
                ix = pk & maskidx
                w = plsc.bitcast(pk & maskw, jnp.float32)
                g0 = plsc.load_gather(vbuf, [ix + voff0])
                g1 = plsc.load_gather(vbuf, [ix + voff1])
                pk2 = plsc.load_gather(packb, [qv32 + (k + 1)])
                ix2 = pk2 & maskidx
                w2 = plsc.bitcast(pk2 & maskw, jnp.float32)
                h0 = plsc.load_gather(vbuf, [ix2 + voff0])
                h1 = plsc.load_gather(vbuf, [ix2 + voff1])
                return (a0 + w * g0, b0 + w2 * h0, a1 + w * g1, b1 + w2 * h1)

            a0, b0, a1, b1 = plsc.parallel_loop(
                0, TOP_K, step=2, unroll=5,
                carry=(zero, zero, zero, zero))(kstep)
            outst[pl.ds(qg * L, L)] = a0 + b0
            outst[pl.ds(HW + qg * L, L)] = a1 + b1
            return 0

        lax.fori_loop(0, HW // L, qgroup, 0, unroll=2)
        pltpu.sync_copy(outst, out_hbm.at[pl.ds((r0 + g * 2) * HW, 2 * HW)])


def _sc_readout(vflat, pack):
    mesh = plsc.VectorSubcoreMesh(core_axis_name="c", subcore_axis_name="s")
    return pl.kernel(
        _readout_tile,
        mesh=mesh,
        compiler_params=pltpu.CompilerParams(needs_layout_passes=False),
        out_type=jax.ShapeDtypeStruct((2 * CV * HW,), jnp.float32),
        scratch_types=[
            pltpu.VMEM((HW * 2 * L,), jnp.int32),
            pltpu.VMEM((4 * T,), jnp.float32),
            pltpu.VMEM((2 * HW,), jnp.float32),
            pltpu.SemaphoreType.DMA,
            pltpu.SemaphoreType.DMA,
        ],
    )(vflat, pack)


@jax.jit
def kernel(q_key, q_selection, mem_key, mem_shrinkage, mem_value):
    qkT = jnp.swapaxes(q_key.reshape(CK, HW), 0, 1)
    qsT = jnp.swapaxes(q_selection.reshape(CK, HW), 0, 1)
    mk = mem_key.reshape(CK, T)
    shr = mem_shrinkage.reshape(1, T)
    vflat = mem_value.reshape(2 * CV, T)

    sim, mrep = _tc_similarity(qkT, qsT, mk, shr)
    pack = _sc_topk(sim, mrep.reshape(HW * L))
    out = _sc_readout(vflat, pack)
    return out.reshape(2, CV, H, W)
